# baseline (speedup 1.0000x reference)
import math

import jax
import jax.numpy as jnp
from jax.experimental import pallas as pl
from jax.experimental.pallas import tpu as pltpu

BSZ = 8
C_LEN = 512
QN = 64
QL = 64
D = 512
H = 512


def _fused_kernel(tags_ref, ctx_ref, q_ref, w1_ref, w2_ref,
                  out_ref, cp_scr, qsum_scr, pos_ref):
    q = pl.program_id(0)
    tcur = tags_ref[q]
    tprev = tags_ref[jnp.maximum(q - 1, 0)]
    seg_start = (q == 0) | (tcur != tprev)

    @pl.when(seg_start)
    def _():
        cp_scr[...] = jax.nn.relu(
            jnp.dot(ctx_ref[0], w1_ref[...], preferred_element_type=jnp.float32)
        ).astype(jnp.bfloat16)
        qsum_scr[...] = jnp.zeros_like(qsum_scr)
        pos_ref[0] = 0

    pos = pos_ref[0]
    inv = 1.0 / jnp.maximum(pos, 1).astype(jnp.float32)
    avg = qsum_scr[...] * inv
    qf = q_ref[0]
    cat = jnp.concatenate(
        [qf.astype(jnp.bfloat16), avg.astype(jnp.bfloat16)], axis=1
    )

    qp = jnp.dot(cat, w2_ref[...], preferred_element_type=jnp.float32)
    qp_b = jax.nn.relu(qp).astype(jnp.bfloat16)

    s = jax.lax.dot_general(
        cp_scr[...], qp_b, (((1,), (1,)), ((), ())),
        preferred_element_type=jnp.float32,
    ) * (1.0 / math.sqrt(H))
    s = s - jnp.max(s, axis=1, keepdims=True)
    e = jnp.exp(s)
    attn = (e / jnp.sum(e, axis=1, keepdims=True)).astype(jnp.bfloat16)

    out_ref[0] = jnp.dot(attn, cat, preferred_element_type=jnp.float32)

    qsum_scr[...] += qf
    pos_ref[0] = pos + 1


def kernel(contexts, questions, tags, W1, W2):
    tags32 = tags.astype(jnp.int32)
    ctx_b = contexts.astype(jnp.bfloat16)
    w1_b = W1.astype(jnp.bfloat16)
    w2_b = W2.astype(jnp.bfloat16)

    out = pl.pallas_call(
        _fused_kernel,
        grid_spec=pltpu.PrefetchScalarGridSpec(
            num_scalar_prefetch=1,
            grid=(QN,),
            in_specs=[
                pl.BlockSpec((1, C_LEN, D), lambda q, t: (t[q], 0, 0)),
                pl.BlockSpec((1, QL, D), lambda q, t: (q, 0, 0)),
                pl.BlockSpec((D, H), lambda q, t: (0, 0)),
                pl.BlockSpec((2 * D, H), lambda q, t: (0, 0)),
            ],
            out_specs=pl.BlockSpec((1, C_LEN, 2 * D), lambda q, t: (q, 0, 0)),
            scratch_shapes=[
                pltpu.VMEM((C_LEN, H), jnp.bfloat16),
                pltpu.VMEM((QL, D), jnp.float32),
                pltpu.SMEM((1,), jnp.int32),
            ],
        ),
        out_shape=jax.ShapeDtypeStruct((QN, C_LEN, 2 * D), jnp.float32),
        compiler_params=pltpu.CompilerParams(dimension_semantics=("arbitrary",)),
    )(tags32, ctx_b, questions, w1_b, w2_b)

    return out
